# baseline (speedup 1.0000x reference)
import functools

import jax
import jax.numpy as jnp
from jax import lax
from jax.experimental import pallas as pl
from jax.experimental.pallas import tpu as pltpu
from jax.experimental.pallas import tpu_sc as plsc

EMBED = 64
NC = 2
NS = 16
NW = NC * NS
CHUNK = 512
SUB = 128


@functools.lru_cache(maxsize=None)
def _gather_call(n_rows: int):
    b_per_w = n_rows // NW
    n_chunks = b_per_w // CHUNK
    mesh = plsc.VectorSubcoreMesh(core_axis_name="c", subcore_axis_name="s")

    @functools.partial(
        pl.kernel,
        out_type=jax.ShapeDtypeStruct((n_rows, EMBED), jnp.float32),
        mesh=mesh,
        scratch_types=[
            pltpu.VMEM((CHUNK,), jnp.int32),
            pltpu.VMEM((CHUNK, EMBED), jnp.float32),
            pltpu.SemaphoreType.DMA,
        ],
        compiler_params=pltpu.CompilerParams(use_tc_tiling_on_sc=False),
    )
    def body(idx_hbm, table_hbm, out_hbm, idx_v, rows_v, sem):
        wid = lax.axis_index("s") * NC + lax.axis_index("c")
        base = wid * b_per_w

        def step(it, carry):
            off = base + it * CHUNK
            pltpu.sync_copy(idx_hbm.at[pl.ds(off, CHUNK)], idx_v)
            copies = [
                pltpu.async_copy(
                    table_hbm.at[idx_v.at[pl.ds(j * SUB, SUB)]],
                    rows_v.at[pl.ds(j * SUB, SUB)],
                    sem,
                )
                for j in range(CHUNK // SUB)
            ]
            for c in copies:
                c.wait()
            pltpu.sync_copy(rows_v, out_hbm.at[pl.ds(off, CHUNK)])
            return carry

        lax.fori_loop(0, n_chunks, step, 0)

    return body


def kernel(x, table):
    batch_shape = x.shape
    flat = x.reshape(-1).astype(jnp.int32)
    out = _gather_call(flat.shape[0])(flat, table)
    return out.reshape(*batch_shape, EMBED)


# speedup vs baseline: 1.0367x; 1.0367x over previous
import functools

import jax
import jax.numpy as jnp
from jax import lax
from jax.experimental import pallas as pl
from jax.experimental.pallas import tpu as pltpu
from jax.experimental.pallas import tpu_sc as plsc

EMBED = 64
NC = 2
NS = 16
NW = NC * NS
RG = 8


@functools.lru_cache(maxsize=None)
def _gather_call(batch: int, seq: int):
    rows_per_w = batch // NW
    n_groups = rows_per_w // RG
    splits = []
    off = 0
    while off < seq:
        n = min(128, seq - off)
        splits.append((off, n))
        off += n
    mesh = plsc.VectorSubcoreMesh(core_axis_name="c", subcore_axis_name="s")

    @functools.partial(
        pl.kernel,
        out_type=jax.ShapeDtypeStruct((batch, seq, EMBED), jnp.float32),
        mesh=mesh,
        scratch_types=[
            pltpu.VMEM((rows_per_w, seq), jnp.int32),
            pltpu.VMEM((RG, seq, EMBED), jnp.float32),
            pltpu.SemaphoreType.DMA,
        ],
        compiler_params=pltpu.CompilerParams(use_tc_tiling_on_sc=False),
    )
    def body(idx_hbm, table_hbm, out_hbm, idx_v, rows_v, sem):
        wid = lax.axis_index("s") * NC + lax.axis_index("c")
        row0 = wid * rows_per_w
        pltpu.sync_copy(idx_hbm.at[pl.ds(row0, rows_per_w)], idx_v)

        def step(g, carry):
            copies = []
            for r in range(RG):
                for (o, n) in splits:
                    copies.append(pltpu.async_copy(
                        table_hbm.at[idx_v.at[g * RG + r, pl.ds(o, n)]],
                        rows_v.at[r, pl.ds(o, n)],
                        sem,
                    ))
            for c in copies:
                c.wait()
            pltpu.sync_copy(rows_v, out_hbm.at[pl.ds(row0 + g * RG, RG)])
            return carry

        lax.fori_loop(0, n_groups, step, 0)

    return body


def kernel(x, table):
    batch, seq = x.shape
    out = _gather_call(batch, seq)(x.astype(jnp.int32), table)
    return out


# speedup vs baseline: 1.0405x; 1.0038x over previous
import functools

import jax
import jax.numpy as jnp
from jax import lax
from jax.experimental import pallas as pl
from jax.experimental.pallas import tpu as pltpu
from jax.experimental.pallas import tpu_sc as plsc

EMBED = 64
NC = 2
NS = 16
NW = NC * NS
CHUNK = 128
G = 5


@functools.lru_cache(maxsize=None)
def _gather_call(n_chunks_total: int):
    chunks_per_w = n_chunks_total // NW
    n_groups = chunks_per_w // G
    n_half = n_groups // 2
    assert n_groups % 2 == 0 and n_groups * G == chunks_per_w

    mesh = plsc.VectorSubcoreMesh(core_axis_name="c", subcore_axis_name="s")

    @functools.partial(
        pl.kernel,
        out_type=jax.ShapeDtypeStruct((n_chunks_total, CHUNK, EMBED), jnp.float32),
        mesh=mesh,
        scratch_types=[
            pltpu.VMEM((chunks_per_w, CHUNK), jnp.int32),
            pltpu.VMEM((G, CHUNK, EMBED), jnp.float32),
            pltpu.VMEM((G, CHUNK, EMBED), jnp.float32),
            pltpu.SemaphoreType.DMA,
            pltpu.SemaphoreType.DMA,
            pltpu.SemaphoreType.DMA,
            pltpu.SemaphoreType.DMA,
        ],
        compiler_params=pltpu.CompilerParams(use_tc_tiling_on_sc=False),
    )
    def body(idx_hbm, table_hbm, out_hbm, idx_v, buf0, buf1, g0, g1, s0, s1):
        wid = lax.axis_index("s") * NC + lax.axis_index("c")
        base = wid * chunks_per_w
        pltpu.sync_copy(idx_hbm.at[pl.ds(base, chunks_per_w)], idx_v)

        def fire_g(buf, sem, grp):
            for j in range(G):
                pltpu.async_copy(table_hbm.at[idx_v.at[grp * G + j]],
                                 buf.at[j], sem)

        def wait_g(buf, sem, grp):
            for j in range(G):
                pltpu.make_async_copy(table_hbm.at[idx_v.at[grp * G + j]],
                                      buf.at[j], sem).wait()

        def fire_s(buf, sem, grp):
            pltpu.async_copy(buf, out_hbm.at[pl.ds(base + grp * G, G)], sem)

        def wait_s(buf, sem, grp):
            pltpu.make_async_copy(buf, out_hbm.at[pl.ds(base + grp * G, G)],
                                  sem).wait()

        fire_g(buf0, g0, 0)

        def step(p, carry):
            wait_g(buf0, g0, 2 * p)

            @pl.when(p > 0)
            def _():
                wait_s(buf1, s1, 2 * p - 1)

            fire_g(buf1, g1, 2 * p + 1)
            fire_s(buf0, s0, 2 * p)

            wait_g(buf1, g1, 2 * p + 1)
            wait_s(buf0, s0, 2 * p)

            @pl.when(p < n_half - 1)
            def _():
                fire_g(buf0, g0, 2 * p + 2)

            fire_s(buf1, s1, 2 * p + 1)
            return carry

        lax.fori_loop(0, n_half, step, 0)
        wait_s(buf1, s1, n_groups - 1)

    return body


def kernel(x, table):
    batch, seq = x.shape
    n_chunks = (batch * seq) // CHUNK
    xf = x.astype(jnp.int32).reshape(n_chunks, CHUNK)
    out = _gather_call(n_chunks)(xf, table)
    return out.reshape(batch, seq, EMBED)


# speedup vs baseline: 1.2289x; 1.1810x over previous
import functools

import jax
import jax.numpy as jnp
from jax import lax
from jax.experimental import pallas as pl
from jax.experimental.pallas import tpu as pltpu
from jax.experimental.pallas import tpu_sc as plsc

EMBED = 64
PADW = 128
NC = 2
NS = 16
NW = NC * NS
CHUNK = 128
G = 2


@functools.lru_cache(maxsize=None)
def _gather_call(n_chunks_total: int):
    chunks_per_w = n_chunks_total // NW
    n_groups = chunks_per_w // G
    n_half = n_groups // 2
    assert n_groups % 2 == 0 and n_groups * G == chunks_per_w

    mesh = plsc.VectorSubcoreMesh(core_axis_name="c", subcore_axis_name="s")

    @functools.partial(
        pl.kernel,
        out_type=jax.ShapeDtypeStruct((n_chunks_total, CHUNK, PADW),
                                      jnp.float32),
        mesh=mesh,
        scratch_types=[
            pltpu.VMEM((chunks_per_w, CHUNK), jnp.int32),
            pltpu.VMEM((G, CHUNK, PADW), jnp.float32),
            pltpu.VMEM((G, CHUNK, PADW), jnp.float32),
            pltpu.SemaphoreType.DMA,
            pltpu.SemaphoreType.DMA,
            pltpu.SemaphoreType.DMA,
            pltpu.SemaphoreType.DMA,
        ],
        compiler_params=pltpu.CompilerParams(use_tc_tiling_on_sc=False),
    )
    def body(idx_hbm, table_hbm, out_hbm, idx_v, buf0, buf1, g0, g1, s0, s1):
        wid = lax.axis_index("s") * NC + lax.axis_index("c")
        base = wid * chunks_per_w
        pltpu.sync_copy(idx_hbm.at[pl.ds(base, chunks_per_w)], idx_v)

        def fire_g(buf, sem, grp):
            for j in range(G):
                pltpu.async_copy(table_hbm.at[idx_v.at[grp * G + j]],
                                 buf.at[j], sem)

        def wait_g(buf, sem, grp):
            for j in range(G):
                pltpu.make_async_copy(table_hbm.at[idx_v.at[grp * G + j]],
                                      buf.at[j], sem).wait()

        def fire_s(buf, sem, grp):
            pltpu.async_copy(buf, out_hbm.at[pl.ds(base + grp * G, G)], sem)

        def wait_s(buf, sem, grp):
            pltpu.make_async_copy(buf, out_hbm.at[pl.ds(base + grp * G, G)],
                                  sem).wait()

        fire_g(buf0, g0, 0)

        def step(p, carry):
            wait_g(buf0, g0, 2 * p)

            @pl.when(p > 0)
            def _():
                wait_s(buf1, s1, 2 * p - 1)

            fire_g(buf1, g1, 2 * p + 1)
            fire_s(buf0, s0, 2 * p)

            wait_g(buf1, g1, 2 * p + 1)
            wait_s(buf0, s0, 2 * p)

            @pl.when(p < n_half - 1)
            def _():
                fire_g(buf0, g0, 2 * p + 2)

            fire_s(buf1, s1, 2 * p + 1)
            return carry

        lax.fori_loop(0, n_half, step, 0)
        wait_s(buf1, s1, n_groups - 1)

    return body


def _tableprep_call(vocab: int):
    vblk = 2048

    def body(in_ref, out_ref):
        out_ref[:, 0:EMBED] = jnp.transpose(in_ref[...], (1, 0))
        out_ref[:, EMBED:PADW] = jnp.zeros((vblk, PADW - EMBED), jnp.float32)

    return pl.pallas_call(
        body,
        grid=(pl.cdiv(vocab, vblk),),
        in_specs=[pl.BlockSpec((EMBED, vblk), lambda i: (0, i))],
        out_specs=pl.BlockSpec((vblk, PADW), lambda i: (i, 0)),
        out_shape=jax.ShapeDtypeStruct((vocab, PADW), jnp.float32),
    )


def _relayout_call(batch: int, seq: int):
    def body(in_ref, out_ref):
        out_ref[0] = jnp.transpose(in_ref[:, 0:EMBED], (1, 0))

    return pl.pallas_call(
        body,
        grid=(seq,),
        in_specs=[pl.BlockSpec((batch, PADW), lambda i: (i, 0))],
        out_specs=pl.BlockSpec((1, EMBED, batch), lambda i: (i, 0, 0)),
        out_shape=jax.ShapeDtypeStruct((seq, EMBED, batch), jnp.float32),
    )


def kernel(x, table):
    batch, seq = x.shape
    vocab, embed = table.shape
    n_chunks = (batch * seq) // CHUNK
    t2 = _tableprep_call(vocab)(table.T)
    xf = x.astype(jnp.int32).T.reshape(n_chunks, CHUNK)
    flat = _gather_call(n_chunks)(xf, t2)
    flat2 = flat.reshape(batch * seq, PADW)
    out_sdb = _relayout_call(batch, seq)(flat2)
    return out_sdb.transpose(2, 0, 1)
